# Initial kernel scaffold; baseline (speedup 1.0000x reference)
import functools

import jax
import jax.numpy as jnp
from jax import lax
from jax.experimental import pallas as pl
from jax.experimental.pallas import tpu as pltpu
from jax.experimental.pallas import tpu_sc as plsc

_N_ATOMS = 1_000_000
_N_TYPES = 119
_N_PROPS = 128
_N_SYS = 1024

_NC = 2
_NS = 16
_NW = _NC * _NS
_L = 16

_CHUNK = 2048
_NCHUNK = -(-_N_ATOMS // _CHUNK)
_HIST = _N_SYS * 128
_HISTP = _HIST + _CHUNK
_ZBLKS = _HISTP // _CHUNK
_DUMMY = _HIST
_OUT_SLICE = _HIST // _NS


def _hist_body(t2i_hbm, types_hbm, sys_hbm, out_hbm,
               t2i_v, types_v, sys_v, keys_v, ones_v, hist_s):
    c = lax.axis_index("c")
    s = lax.axis_index("s")
    wid = s * _NC + c

    pltpu.sync_copy(t2i_hbm, t2i_v)
    for i in range(8):
        ones_v[pl.ds(i * _L, _L)] = jnp.full((_L,), 1.0, jnp.float32)

    def _zbuf(i, _):
        types_v[pl.ds(i * _L, _L)] = jnp.zeros((_L,), jnp.int32)
        return 0
    lax.fori_loop(0, _CHUNK // _L, _zbuf, 0)
    for rep in range(-(-_ZBLKS // _NS)):
        blk = s + rep * _NS

        @pl.when(blk < _ZBLKS)
        def _():
            pltpu.sync_copy(types_v, hist_s.at[pl.ds(blk * _CHUNK, _CHUNK)])
    plsc.subcore_barrier()

    nchunks_w = (_NCHUNK - 1 - wid) // _NW + 1

    def _chunk(k, _):
        cid = wid + k * _NW
        base = jnp.minimum(cid * _CHUNK, _N_ATOMS - _CHUNK)
        start_lane = cid * _CHUNK - base
        pltpu.sync_copy(types_hbm.at[pl.ds(base, _CHUNK)], types_v)
        pltpu.sync_copy(sys_hbm.at[pl.ds(base, _CHUNK)], sys_v)

        def _group(g, _):
            t16 = types_v[pl.ds(g * _L, _L)]
            s16 = sys_v[pl.ds(g * _L, _L)]
            r16 = plsc.load_gather(t2i_v, [t16])
            key = jnp.bitwise_or(lax.shift_left(s16, 7), r16)
            lane = g * _L + lax.iota(jnp.int32, 16)
            key = jnp.where(lane >= start_lane, key, _DUMMY)
            row = lax.shift_right_logical(g, 3)
            col = lax.shift_left(jnp.bitwise_and(g, 7), 4)
            keys_v[row, pl.ds(col, _L)] = key
            return 0
        lax.fori_loop(0, _CHUNK // _L, _group, 0)

        for j in range(_CHUNK // 128):
            pltpu.sync_copy(ones_v, hist_s.at[keys_v.at[j]], add=True)
        return 0
    lax.fori_loop(0, nchunks_w, _chunk, 0)

    plsc.subcore_barrier()
    pltpu.sync_copy(hist_s.at[pl.ds(s * _OUT_SLICE, _OUT_SLICE)],
                    out_hbm.at[c, s])


_hist_kernel = functools.partial(
    pl.kernel,
    out_type=jax.ShapeDtypeStruct((_NC, _NS, _OUT_SLICE), jnp.float32),
    mesh=plsc.VectorSubcoreMesh(
        core_axis_name="c", subcore_axis_name="s",
        num_cores=_NC, num_subcores=_NS),
    scratch_types=[
        pltpu.VMEM((128,), jnp.int32),
        pltpu.VMEM((_CHUNK,), jnp.int32),
        pltpu.VMEM((_CHUNK,), jnp.int32),
        pltpu.VMEM((_CHUNK // 128, 128), jnp.int32),
        pltpu.VMEM((128,), jnp.float32),
        pltpu.VMEM_SHARED((_HISTP,), jnp.float32),
    ],
)(_hist_body)


def _mm_body(h_ref, w_ref, o_ref):
    counts = h_ref[0] + h_ref[1]
    o_ref[...] = jnp.dot(counts, w_ref[...], preferred_element_type=jnp.float32)


def _mm(hist2, w_pad):
    return pl.pallas_call(
        _mm_body,
        out_shape=jax.ShapeDtypeStruct((_N_SYS, _N_PROPS), jnp.float32),
    )(hist2, w_pad)


def kernel(weights, types, system_indices, type_to_index):
    t2i_pad = jnp.zeros((128,), jnp.int32).at[:_N_TYPES].set(type_to_index)
    w_pad = jnp.zeros((128, _N_PROPS), jnp.float32).at[:_N_TYPES].set(weights)
    hist = _hist_kernel(t2i_pad, types, system_indices)
    return _mm(hist.reshape(_NC, _N_SYS, 128), w_pad)


# speedup vs baseline: 115.3298x; 115.3298x over previous
import functools

import jax
import jax.numpy as jnp
from jax import lax
from jax.experimental import pallas as pl
from jax.experimental.pallas import tpu as pltpu
from jax.experimental.pallas import tpu_sc as plsc

_N_ATOMS = 1_000_000
_N_TYPES = 119
_N_PROPS = 128
_N_SYS = 1024

_NC = 2
_NS = 16
_NW = _NC * _NS
_L = 16

_CHUNK = 2048
_NCHUNK = -(-_N_ATOMS // _CHUNK)
_HIST = _N_SYS * 128
_HISTP = _HIST + _CHUNK
_ZBLKS = _HISTP // _CHUNK
_DUMMY = _HIST
_OUT_SLICE = _HIST // _NS


def _hist_body(t2i_hbm, types_hbm, sys_hbm, out_hbm,
               t2i_v, types_v, sys_v, keys_v, ones_v, zeros_v, hist_s):
    c = lax.axis_index("c")
    s = lax.axis_index("s")
    wid = s * _NC + c

    pltpu.sync_copy(t2i_hbm, t2i_v)
    for i in range(8):
        ones_v[pl.ds(i * _L, _L)] = jnp.full((_L,), 1.0, jnp.float32)

    def _zbuf(i, _):
        zeros_v[pl.ds(i * _L, _L)] = jnp.zeros((_L,), jnp.float32)
        return 0
    lax.fori_loop(0, _CHUNK // _L, _zbuf, 0)
    for rep in range(-(-_ZBLKS // _NS)):
        blk = s + rep * _NS

        @pl.when(blk < _ZBLKS)
        def _():
            pltpu.sync_copy(zeros_v, hist_s.at[pl.ds(blk * _CHUNK, _CHUNK)])
    plsc.subcore_barrier()

    nchunks_w = (_NCHUNK - 1 - wid) // _NW + 1

    def _chunk(k, _):
        cid = wid + k * _NW
        base = jnp.minimum(cid * _CHUNK, _N_ATOMS - _CHUNK)
        start_lane = cid * _CHUNK - base
        pltpu.sync_copy(types_hbm.at[pl.ds(base, _CHUNK)], types_v)
        pltpu.sync_copy(sys_hbm.at[pl.ds(base, _CHUNK)], sys_v)

        def _group(g, _):
            t16 = types_v[pl.ds(g * _L, _L)]
            s16 = sys_v[pl.ds(g * _L, _L)]
            r16 = plsc.load_gather(t2i_v, [t16])
            key = jnp.bitwise_or(lax.shift_left(s16, 7), r16)
            lane = g * _L + lax.iota(jnp.int32, 16)
            key = jnp.where(lane >= start_lane, key, _DUMMY)
            row = lax.shift_right_logical(g, 3)
            col = lax.shift_left(jnp.bitwise_and(g, 7), 4)
            keys_v[row, pl.ds(col, _L)] = key
            return 0
        lax.fori_loop(0, _CHUNK // _L, _group, 0)

        for j in range(_CHUNK // 128):
            pltpu.sync_copy(ones_v, hist_s.at[keys_v.at[j]], add=True)
        return 0
    lax.fori_loop(0, nchunks_w, _chunk, 0)

    plsc.subcore_barrier()
    pltpu.sync_copy(hist_s.at[pl.ds(s * _OUT_SLICE, _OUT_SLICE)],
                    out_hbm.at[c, s])


_hist_kernel = functools.partial(
    pl.kernel,
    out_type=jax.ShapeDtypeStruct((_NC, _NS, _OUT_SLICE), jnp.float32),
    mesh=plsc.VectorSubcoreMesh(
        core_axis_name="c", subcore_axis_name="s",
        num_cores=_NC, num_subcores=_NS),
    scratch_types=[
        pltpu.VMEM((128,), jnp.int32),
        pltpu.VMEM((_CHUNK,), jnp.int32),
        pltpu.VMEM((_CHUNK,), jnp.int32),
        pltpu.VMEM((_CHUNK // 128, 128), jnp.int32),
        pltpu.VMEM((128,), jnp.float32),
        pltpu.VMEM((_CHUNK,), jnp.float32),
        pltpu.VMEM_SHARED((_HISTP,), jnp.float32),
    ],
    compiler_params=pltpu.CompilerParams(needs_layout_passes=False),
)(_hist_body)


def _mm_body(h_ref, w_ref, o_ref):
    counts = h_ref[0] + h_ref[1]
    o_ref[...] = jnp.dot(counts, w_ref[...], preferred_element_type=jnp.float32)


def _mm(hist2, w_pad):
    return pl.pallas_call(
        _mm_body,
        out_shape=jax.ShapeDtypeStruct((_N_SYS, _N_PROPS), jnp.float32),
    )(hist2, w_pad)


def kernel(weights, types, system_indices, type_to_index):
    t2i_pad = jnp.zeros((128,), jnp.int32).at[:_N_TYPES].set(type_to_index)
    w_pad = jnp.zeros((128, _N_PROPS), jnp.float32).at[:_N_TYPES].set(weights)
    hist = _hist_kernel(t2i_pad, types, system_indices)
    return _mm(hist.reshape(_NC, _N_SYS, 128), w_pad)
